# Initial kernel scaffold; baseline (speedup 1.0000x reference)
import jax
import jax.numpy as jnp
from jax.experimental import pallas as pl


def kernel(node_features, edge_index, edge_feat, eig_vec, W_in, b_in, ln_in_g, ln_in_b, W_M, b_M, W_U, b_U, bn_g, bn_b, W_mix, b_mix, ln_int_g, ln_int_b, res_scale, W_o1, b_o1, ln_o_g, ln_o_b, W_o2, b_o2):
    raise NotImplementedError("write your pallas kernel here")


# speedup vs baseline: 1.9511x; 1.9511x over previous
import functools
import jax
import jax.numpy as jnp
from jax import lax
from jax.experimental import pallas as pl
from jax.experimental.pallas import tpu as pltpu
from jax.experimental.pallas import tpu_sc as plsc

N = 10000
E = 160000
H = 128
HH = 64
NP = 10240
NPH = 3456
NP3 = 3 * NPH
R2 = 216
ACC_ROWS = 3472
DUMP = 3456
KB = 128
CH = 4000
NEG = -3.0e38
DELTA = 1.0

def _splat_i32(v):
    return jnp.full((16,), v, jnp.int32)


def _sc_pass0_body(dst_h, src_h, eig1_h, out_h, eig1_v, accd, acca, accw,
                   dstb, srcb):
    c = lax.axis_index("c")
    s = lax.axis_index("s")

    @pl.when(c == 0)
    def _work():
        pltpu.sync_copy(eig1_h, eig1_v)
        l0 = jnp.where(lax.iota(jnp.int32, 16) == 0, 1.0, 0.0)

        def _z(r, _):
            z = jnp.zeros((16,), jnp.float32)
            accd[pl.ds(r * 16, 16)] = z
            acca[pl.ds(r * 16, 16)] = z
            accw[pl.ds(r * 16, 16)] = z
            return 0
        lax.fori_loop(0, NP // 16, _z, 0)

        base = s * (E // 16)

        def _chunk(k, _):
            pltpu.sync_copy(dst_h.at[pl.ds(base + k * 2000, 2000)], dstb)
            pltpu.sync_copy(src_h.at[pl.ds(base + k * 2000, 2000)], srcb)

            def _grp(g, _):
                d = dstb[pl.ds(g * 16, 16)]
                sv = srcb[pl.ds(g * 16, 16)]
                e1d = plsc.load_gather(eig1_v, [d])
                e1s = plsc.load_gather(eig1_v, [sv])
                dw = e1d - e1s
                adw = jnp.abs(dw)
                for j in range(16):
                    dj = d[j]
                    sl = pl.ds(dj, 16)
                    accd[sl] = accd[sl] + l0
                    acca[sl] = acca[sl] + l0 * adw[j]
                    accw[sl] = accw[sl] + l0 * dw[j]
                return 0
            lax.fori_loop(0, 2000 // 16, _grp, 0)
            return 0
        lax.fori_loop(0, (E // 16) // 2000, _chunk, 0)

        for a, acc in enumerate((accd, acca, accw)):
            pltpu.sync_copy(acc, out_h.at[pl.ds((a * 16 + s) * NP, NP)])


@functools.cache
def _build_pass0():
    mesh = plsc.VectorSubcoreMesh(core_axis_name="c", subcore_axis_name="s")

    @functools.partial(
        pl.kernel, mesh=mesh,
        compiler_params=pltpu.CompilerParams(needs_layout_passes=False, use_tc_tiling_on_sc=False),
        out_type=jax.ShapeDtypeStruct((48 * NP,), jnp.float32),
        scratch_types=[
            pltpu.VMEM((NP,), jnp.float32),
            pltpu.VMEM((NP,), jnp.float32),
            pltpu.VMEM((NP,), jnp.float32),
            pltpu.VMEM((NP,), jnp.float32),
            pltpu.VMEM((2000,), jnp.int32),
            pltpu.VMEM((2000,), jnp.int32),
        ],
    )
    def k(dst_h, src_h, eig1_h, out_h, *rest):
        _sc_pass0_body(dst_h, src_h, eig1_h, out_h, *rest)

    return k


def _sc_pass0(dst, src, eig1):
    return _build_pass0()(dst, src, eig1)


def _sc_edge_body(A2_h, C2_h, src_h, dst_h, eig1_h, rabs_h,
                  s0_h, s1_h, s2_h, smax_h,
                  eig1_v, rabs_v, maxacc, dstb, srcb,
                  dstl, srcl, eidl, avl, dxl,
                  bufA, bufC, bufAv, bufDx,
                  acc0, acc1, acc2, sem):
    c = lax.axis_index("c")
    s = lax.axis_index("s")

    pltpu.sync_copy(eig1_h, eig1_v)
    pltpu.sync_copy(rabs_h, rabs_v)

    def _flush(fill):
        pltpu.async_copy(A2_h.at[srcl], bufA, sem).wait()
        pltpu.async_copy(C2_h.at[eidl], bufC, sem).wait()

        def _r(r, _):
            rs = jnp.full((16,), r, jnp.int32)
            loc = plsc.load_gather(dstl, [rs])[0] - s * R2
            avr = plsc.load_gather(avl, [rs])
            dxr = plsc.load_gather(dxl, [rs])
            for j in range(4):
                sl = pl.ds(j * 16, 16)
                mj = bufA[r, sl] + bufC[r, sl]
                bufA[r, sl] = mj
                bufAv[r, sl] = mj * avr
                bufDx[r, sl] = mj * dxr
                maxacc[loc, sl] = jnp.maximum(maxacc[loc, sl], mj)
            return 0
        lax.fori_loop(0, fill, _r, 0)

        pltpu.sync_copy(bufA, acc0.at[dstl], add=True)
        pltpu.sync_copy(bufAv, acc1.at[dstl], add=True)
        pltpu.sync_copy(bufDx, acc2.at[dstl], add=True)
        for q in range(KB // 16):
            dstl[pl.ds(q * 16, 16)] = _splat_i32(DUMP)

    for p in range(3):
        plo = p * NPH
        tlo = plo + s * R2
        thi = tlo + R2

        def _initm(r, _):
            for j in range(4):
                maxacc[r, pl.ds(j * 16, 16)] = jnp.full((16,), NEG,
                                                        jnp.float32)
            return 0
        lax.fori_loop(0, R2, _initm, 0)

        def _zb(r, _):
            for j in range(4):
                bufA[r, pl.ds(j * 16, 16)] = jnp.zeros((16,), jnp.float32)
            return 0
        lax.fori_loop(0, KB, _zb, 0)

        zlo = s * (ACC_ROWS // 16)
        for acc in (acc0, acc1, acc2):
            pltpu.sync_copy(bufA, acc.at[pl.ds(zlo, KB)])
            pltpu.sync_copy(bufA.at[pl.ds(0, 89)],
                            acc.at[pl.ds(zlo + KB, 89)])

        for q in range(KB // 16):
            dstl[pl.ds(q * 16, 16)] = _splat_i32(DUMP)
            srcl[pl.ds(q * 16, 16)] = _splat_i32(0)
            eidl[pl.ds(q * 16, 16)] = _splat_i32(0)

        plsc.subcore_barrier()

        def _chunk(k, fill):
            pltpu.sync_copy(dst_h.at[pl.ds(k * CH, CH)], dstb)
            pltpu.sync_copy(src_h.at[pl.ds(k * CH, CH)], srcb)

            def _grp(g, fill):
                d = dstb[pl.ds(g * 16, 16)]
                msk = (d >= tlo) & (d < thi)
                cnt = jnp.reshape(plsc.all_reduce_population_count(msk),
                                  (-1,))[0]

                def _do(f):
                    sv = srcb[pl.ds(g * 16, 16)]
                    e1d = plsc.load_gather(eig1_v, [d])
                    e1s = plsc.load_gather(eig1_v, [sv])
                    dw = e1d - e1s
                    ab = plsc.load_gather(rabs_v, [d]) + 1e-30
                    av = jnp.abs(dw) / ab
                    dx = dw / ab
                    eid = k * CH + g * 16 + lax.iota(jnp.int32, 16)
                    fs = pl.ds(f, 16)
                    plsc.store_compressed(dstl.at[fs], d - plo, mask=msk)
                    plsc.store_compressed(srcl.at[fs], sv * 2 + c, mask=msk)
                    plsc.store_compressed(eidl.at[fs], eid * 2 + c, mask=msk)
                    plsc.store_compressed(avl.at[fs], av, mask=msk)
                    plsc.store_compressed(dxl.at[fs], dx, mask=msk)
                    return f + cnt

                fill = lax.cond(cnt > 0, _do, lambda f: f, fill)

                def _fl(f):
                    _flush(f)
                    return 0
                return lax.cond(fill > KB - 16, _fl, lambda f: f, fill)

            return lax.fori_loop(0, CH // 16, _grp, fill)

        fill = lax.fori_loop(0, E // CH, _chunk, 0)

        def _fl2(f):
            _flush(f)
            return 0
        lax.cond(fill > 0, _fl2, lambda f: f, fill)

        plsc.subcore_barrier()

        obase = c * NP3 + plo + s * R2
        for acc, out in ((acc0, s0_h), (acc1, s1_h), (acc2, s2_h)):
            pltpu.sync_copy(acc.at[pl.ds(s * R2, KB)], bufC)
            pltpu.sync_copy(bufC, out.at[pl.ds(obase, KB)])
            pltpu.sync_copy(acc.at[pl.ds(s * R2 + KB, 88)],
                            bufC.at[pl.ds(0, 88)])
            pltpu.sync_copy(bufC.at[pl.ds(0, 88)],
                            out.at[pl.ds(obase + KB, 88)])
        pltpu.sync_copy(maxacc, smax_h.at[pl.ds(obase, R2)])

        plsc.subcore_barrier()


_sc_out4 = tuple(jax.ShapeDtypeStruct((2 * NP3, HH), jnp.float32)
                 for _ in range(4))


@functools.cache
def _build_edge():
    mesh = plsc.VectorSubcoreMesh(core_axis_name="c", subcore_axis_name="s")

    @functools.partial(
        pl.kernel, mesh=mesh,
        compiler_params=pltpu.CompilerParams(needs_layout_passes=False, use_tc_tiling_on_sc=False),
        out_type=_sc_out4,
        scratch_types=[
        pltpu.VMEM((NP,), jnp.float32),
        pltpu.VMEM((NP,), jnp.float32),
        pltpu.VMEM((R2, HH), jnp.float32),
        pltpu.VMEM((CH,), jnp.int32),
        pltpu.VMEM((CH,), jnp.int32),
        pltpu.VMEM((KB,), jnp.int32),
        pltpu.VMEM((KB,), jnp.int32),
        pltpu.VMEM((KB,), jnp.int32),
        pltpu.VMEM((KB,), jnp.float32),
        pltpu.VMEM((KB,), jnp.float32),
        pltpu.VMEM((KB, HH), jnp.float32),
        pltpu.VMEM((KB, HH), jnp.float32),
        pltpu.VMEM((KB, HH), jnp.float32),
        pltpu.VMEM((KB, HH), jnp.float32),
            pltpu.VMEM_SHARED((ACC_ROWS, HH), jnp.float32),
            pltpu.VMEM_SHARED((ACC_ROWS, HH), jnp.float32),
            pltpu.VMEM_SHARED((ACC_ROWS, HH), jnp.float32),
            pltpu.SemaphoreType.DMA,
        ],
    )
    def k(A2_h, C2_h, src_h, dst_h, eig1_h, rabs_h, s0, s1, s2, sm, *rest):
        _sc_edge_body(A2_h, C2_h, src_h, dst_h, eig1_h, rabs_h, s0, s1, s2,
                      sm, *rest)

    return k


def _sc_edge(A2, C2, src, dst, eig1, rabs):
    return _build_edge()(A2, C2, src, dst, eig1, rabs)


BR = 1000


def _ln(x, g, b):
    m = jnp.mean(x, axis=-1, keepdims=True)
    v = jnp.mean((x - m) ** 2, axis=-1, keepdims=True)
    return (x - m) * lax.rsqrt(v + 1e-5) * g + b


def _mm(x, w):
    return lax.dot_general(x, w, (((1,), (1,)), ((), ())),
                           preferred_element_type=jnp.float32)


def _tk_stats_body(x_r, o_r):
    x = x_r[...]
    rows = [jnp.sum(x[a * 16:(a + 1) * 16], axis=0, keepdims=True)
            for a in range(3)]
    o_r[...] = jnp.concatenate(rows + [jnp.zeros((5, x.shape[1]),
                                                 jnp.float32)], axis=0)


def _tk_stats(x):
    return pl.pallas_call(
        _tk_stats_body,
        grid=(NP // 1024,),
        in_specs=[pl.BlockSpec((48, 1024), lambda i: (0, i))],
        out_specs=pl.BlockSpec((8, 1024), lambda i: (0, i)),
        out_shape=jax.ShapeDtypeStruct((8, NP), jnp.float32),
    )(x)


def _tk_in_body(x_r, w_r, b_r, g_r, bb_r, o_r):
    h = _mm(x_r[...], w_r[...]) + b_r[...]
    o_r[...] = jnp.maximum(_ln(h, g_r[...], bb_r[...]), 0.0)


def _tk_in(x, w, b, g, bb):
    return pl.pallas_call(
        _tk_in_body,
        grid=(N // BR,),
        in_specs=[
            pl.BlockSpec((BR, H), lambda i: (i, 0)),
            pl.BlockSpec((H, H), lambda i: (0, 0)),
            pl.BlockSpec((1, H), lambda i: (0, 0)),
            pl.BlockSpec((1, H), lambda i: (0, 0)),
            pl.BlockSpec((1, H), lambda i: (0, 0)),
        ],
        out_specs=pl.BlockSpec((BR, H), lambda i: (i, 0)),
        out_shape=jax.ShapeDtypeStruct((N, H), jnp.float32),
    )(x, w, b, g, bb)


def _tk_ab_body(h_r, ws_r, wd_r, a_r, b_r):
    a_r[...] = _mm(h_r[...], ws_r[...])
    b_r[...] = _mm(h_r[...], wd_r[...])


def _tk_ab(h, ws, wd):
    return pl.pallas_call(
        _tk_ab_body,
        grid=(N // BR,),
        in_specs=[
            pl.BlockSpec((BR, H), lambda i: (i, 0)),
            pl.BlockSpec((H, H), lambda i: (0, 0)),
            pl.BlockSpec((H, H), lambda i: (0, 0)),
        ],
        out_specs=[
            pl.BlockSpec((BR, H), lambda i: (i, 0)),
            pl.BlockSpec((BR, H), lambda i: (i, 0)),
        ],
        out_shape=[jax.ShapeDtypeStruct((N, H), jnp.float32),
                   jax.ShapeDtypeStruct((N, H), jnp.float32)],
    )(h, ws, wd)


ER = 2000


def _tk_c_body(ef_r, w0_r, b0_r, w1_r, b1_r, c0_r, c1_r):
    ef = ef_r[...]
    c0_r[...] = _mm(ef, w0_r[...]) + b0_r[...]
    c1_r[...] = _mm(ef, w1_r[...]) + b1_r[...]


def _tk_c(ef, w0, b0, w1, b1):
    return pl.pallas_call(
        _tk_c_body,
        grid=(E // ER,),
        in_specs=[
            pl.BlockSpec((ER, 16), lambda i: (i, 0)),
            pl.BlockSpec((H, 16), lambda i: (0, 0)),
            pl.BlockSpec((1, H), lambda i: (0, 0)),
            pl.BlockSpec((H, 16), lambda i: (0, 0)),
            pl.BlockSpec((1, H), lambda i: (0, 0)),
        ],
        out_specs=[
            pl.BlockSpec((ER, H), lambda i: (i, 0)),
            pl.BlockSpec((ER, H), lambda i: (i, 0)),
        ],
        out_shape=[jax.ShapeDtypeStruct((E, H), jnp.float32),
                   jax.ShapeDtypeStruct((E, H), jnp.float32)],
    )(ef, w0, b0, w1, b1)


def _tk_tower_body(h_r, b_r, s0_r, s1_r, s2_r, sm_r, st_r, wu_r, bu_r, o_r):
    h = h_r[...]
    B = b_r[...]
    s0 = jnp.concatenate([s0_r[0], s0_r[1]], axis=-1)
    s1 = jnp.concatenate([s1_r[0], s1_r[1]], axis=-1)
    s2 = jnp.concatenate([s2_r[0], s2_r[1]], axis=-1)
    smx = jnp.concatenate([sm_r[0], sm_r[1]], axis=-1)
    st = st_r[...]
    deg = st[:, 0:1]
    rabs = st[:, 1:2]
    rdw = st[:, 2:3]
    degs = jnp.maximum(deg, 1.0)
    absf = rabs + 1e-30
    sumav = rabs / absf
    sumdx = rdw / absf
    s_sum = s0 + deg * B
    s_mean = s_sum / degs
    s_max = jnp.where(deg > 0.0, smx + B, 0.0)
    s_av = s1 + sumav * B
    s_dx = jnp.abs(s2 + sumdx * B - h * sumdx)
    logd = jnp.log(degs + 1.0)
    amp = logd / DELTA
    att = DELTA / logd
    wu = wu_r[...]
    aggs = (s_mean, s_max, s_sum, s_av, s_dx)
    p0 = _mm(h, wu[:, 0:H])
    for a in range(5):
        p0 = p0 + _mm(aggs[a], wu[:, (1 + a) * H:(2 + a) * H])
    p1 = _mm(aggs[0], wu[:, 6 * H:7 * H])
    p2 = _mm(aggs[0], wu[:, 11 * H:12 * H])
    for a in range(1, 5):
        p1 = p1 + _mm(aggs[a], wu[:, (6 + a) * H:(7 + a) * H])
        p2 = p2 + _mm(aggs[a], wu[:, (11 + a) * H:(12 + a) * H])
    o_r[...] = p0 + amp * p1 + att * p2 + bu_r[...]


def _tk_tower(h, B, s0, s1, s2, sm, st, wu, bu):
    half = pl.BlockSpec((2, BR, HH), lambda i: (0, i, 0))
    return pl.pallas_call(
        _tk_tower_body,
        grid=(N // BR,),
        in_specs=[
            pl.BlockSpec((BR, H), lambda i: (i, 0)),
            pl.BlockSpec((BR, H), lambda i: (i, 0)),
            half, half, half, half,
            pl.BlockSpec((BR, 8), lambda i: (i, 0)),
            pl.BlockSpec((H, 16 * H), lambda i: (0, 0)),
            pl.BlockSpec((1, H), lambda i: (0, 0)),
        ],
        out_specs=pl.BlockSpec((BR, H), lambda i: (i, 0)),
        out_shape=jax.ShapeDtypeStruct((N, H), jnp.float32),
    )(h, B, s0, s1, s2, sm, st, wu, bu)


def _tk_post_body(tw_r, h_r, wm_r, bm_r, bg_r, bb_r, lg_r, lb_r, rs_r, o_r,
                  stats):
    i = pl.program_id(0)
    phase = i // (N // BR)

    @pl.when(phase == 0)
    def _p0():
        @pl.when(i == 0)
        def _z():
            stats[...] = jnp.zeros((8, H), jnp.float32)
        tw = tw_r[...]
        stats[0:1, :] = stats[0:1, :] + jnp.sum(tw, axis=0, keepdims=True)
        stats[1:2, :] = stats[1:2, :] + jnp.sum(tw * tw, axis=0, keepdims=True)

    @pl.when(phase == 1)
    def _p1():
        tw = tw_r[...]
        h = h_r[...]
        mu = stats[0:1, :] / N
        var = stats[1:2, :] / N - mu * mu
        t = (tw - mu) * lax.rsqrt(var + 1e-5) * bg_r[...] + bb_r[...]
        mixed = _mm(t, wm_r[...]) + bm_r[...]
        mixed = jnp.where(mixed > 0.0, mixed, 0.01 * mixed)
        conv = mixed + h
        hn = jnp.maximum(_ln(conv, lg_r[...], lb_r[...]), 0.0)
        gate = 1.0 / (1.0 + jnp.exp(-rs_r[...]))
        o_r[...] = hn + gate * h


def _tk_post(tw, h, wm, bm, bg, bb, lg, lb, rs):
    nb = N // BR
    return pl.pallas_call(
        _tk_post_body,
        grid=(2 * nb,),
        in_specs=[
            pl.BlockSpec((BR, H), lambda i: (i % nb, 0)),
            pl.BlockSpec((BR, H), lambda i: (i % nb, 0)),
            pl.BlockSpec((H, H), lambda i: (0, 0)),
            pl.BlockSpec((1, H), lambda i: (0, 0)),
            pl.BlockSpec((1, H), lambda i: (0, 0)),
            pl.BlockSpec((1, H), lambda i: (0, 0)),
            pl.BlockSpec((1, H), lambda i: (0, 0)),
            pl.BlockSpec((1, H), lambda i: (0, 0)),
            pl.BlockSpec((1, H), lambda i: (0, 0)),
        ],
        out_specs=pl.BlockSpec((BR, H), lambda i: (i % nb, 0)),
        out_shape=jax.ShapeDtypeStruct((N, H), jnp.float32),
        scratch_shapes=[pltpu.VMEM((8, H), jnp.float32)],
    )(tw, h, wm, bm, bg, bb, lg, lb, rs)


def _tk_out_body(h_r, w1_r, b1_r, g_r, bb_r, w2_r, b2_r, o_r):
    t = _mm(h_r[...], w1_r[...]) + b1_r[...]
    t = jnp.maximum(_ln(t, g_r[...], bb_r[...]), 0.0)
    o_r[...] = _mm(t, w2_r[...]) + b2_r[...]


def _tk_out(h, w1, b1, g, bb, w2, b2):
    return pl.pallas_call(
        _tk_out_body,
        grid=(N // BR,),
        in_specs=[
            pl.BlockSpec((BR, H), lambda i: (i, 0)),
            pl.BlockSpec((H, H), lambda i: (0, 0)),
            pl.BlockSpec((1, H), lambda i: (0, 0)),
            pl.BlockSpec((1, H), lambda i: (0, 0)),
            pl.BlockSpec((1, H), lambda i: (0, 0)),
            pl.BlockSpec((H, H), lambda i: (0, 0)),
            pl.BlockSpec((1, H), lambda i: (0, 0)),
        ],
        out_specs=pl.BlockSpec((BR, H), lambda i: (i, 0)),
        out_shape=jax.ShapeDtypeStruct((N, H), jnp.float32),
    )(h, w1, b1, g, bb, w2, b2)


def kernel(node_features, edge_index, edge_feat, eig_vec, W_in, b_in, ln_in_g,
           ln_in_b, W_M, b_M, W_U, b_U, bn_g, bn_b, W_mix, b_mix, ln_int_g,
           ln_int_b, res_scale, W_o1, b_o1, ln_o_g, ln_o_b, W_o2, b_o2):
    src = edge_index[0]
    dst = edge_index[1]
    eig1 = jnp.zeros((NP,), jnp.float32).at[:N].set(eig_vec[:, 1])

    r2 = lambda v: v.reshape(1, H)
    h = _tk_in(node_features, W_in, r2(b_in), r2(ln_in_g), r2(ln_in_b))

    parts = _sc_pass0(dst, src, eig1).reshape(48, NP)
    stats3 = _tk_stats(parts)
    st = jnp.zeros((N, 8), jnp.float32).at[:, 0:3].set(stats3[:3, :N].T)

    C0, C1 = _tk_c(edge_feat, W_M[0][:, 2 * H:], r2(b_M[0]),
                   W_M[1][:, 2 * H:], r2(b_M[1]))
    Cs = (C0, C1)

    for l in range(2):
        A, B = _tk_ab(h, W_M[l][:, :H], W_M[l][:, H:2 * H])
        A2 = A.reshape(2 * N, HH)
        C2 = Cs[l].reshape(2 * E, HH)
        s0, s1, s2, sm = _sc_edge(A2, C2, src, dst, eig1, stats3[1])
        r3 = lambda x: x.reshape(2, NP3, HH)[:, :N]
        tower = _tk_tower(h, B, r3(s0), r3(s1), r3(s2), r3(sm),
                          st, W_U[l], r2(b_U[l]))
        h = _tk_post(tower, h, W_mix[l], r2(b_mix[l]), r2(bn_g[l]),
                     r2(bn_b[l]), r2(ln_int_g[l]), r2(ln_int_b[l]),
                     jnp.full((1, H), res_scale[l], jnp.float32))

    return _tk_out(h, W_o1, r2(b_o1), r2(ln_o_g), r2(ln_o_b), W_o2, r2(b_o2))


# speedup vs baseline: 2.0066x; 1.0284x over previous
import functools
import jax
import jax.numpy as jnp
from jax import lax
from jax.experimental import pallas as pl
from jax.experimental.pallas import tpu as pltpu
from jax.experimental.pallas import tpu_sc as plsc

N = 10000
E = 160000
H = 128
HH = 64
NP = 10240
NPH = 3456
NPHASE = 3
NP3 = NPHASE * NPH
R2 = NPH // 16
ACC_ROWS = NPH + 16
DUMP = NPH
KB = 128
CH = 8000
NCH = E // CH
NEG = -3.0e38
DELTA = 1.0

def _splat_i32(v):
    return jnp.full((16,), v, jnp.int32)


def _sc_pass0_body(dst_h, src_h, eig1_h, out_h, eig1_v, accd, acca, accw,
                   dstb, srcb):
    c = lax.axis_index("c")
    s = lax.axis_index("s")

    @pl.when(c == 0)
    def _work():
        pltpu.sync_copy(eig1_h, eig1_v)
        l0 = jnp.where(lax.iota(jnp.int32, 16) == 0, 1.0, 0.0)

        def _z(r, _):
            z = jnp.zeros((16,), jnp.float32)
            accd[pl.ds(r * 16, 16)] = z
            acca[pl.ds(r * 16, 16)] = z
            accw[pl.ds(r * 16, 16)] = z
            return 0
        lax.fori_loop(0, NP // 16, _z, 0)

        base = s * (E // 16)

        def _chunk(k, _):
            pltpu.sync_copy(dst_h.at[pl.ds(base + k * 2000, 2000)], dstb)
            pltpu.sync_copy(src_h.at[pl.ds(base + k * 2000, 2000)], srcb)

            def _grp(g, _):
                d = dstb[pl.ds(g * 16, 16)]
                sv = srcb[pl.ds(g * 16, 16)]
                e1d = plsc.load_gather(eig1_v, [d])
                e1s = plsc.load_gather(eig1_v, [sv])
                dw = e1d - e1s
                adw = jnp.abs(dw)
                for j in range(16):
                    dj = d[j]
                    sl = pl.ds(dj, 16)
                    accd[sl] = accd[sl] + l0
                    acca[sl] = acca[sl] + l0 * adw[j]
                    accw[sl] = accw[sl] + l0 * dw[j]
                return 0
            lax.fori_loop(0, 2000 // 16, _grp, 0)
            return 0
        lax.fori_loop(0, (E // 16) // 2000, _chunk, 0)

        for a, acc in enumerate((accd, acca, accw)):
            pltpu.sync_copy(acc, out_h.at[pl.ds((a * 16 + s) * NP, NP)])


@functools.cache
def _build_pass0():
    mesh = plsc.VectorSubcoreMesh(core_axis_name="c", subcore_axis_name="s")

    @functools.partial(
        pl.kernel, mesh=mesh,
        compiler_params=pltpu.CompilerParams(needs_layout_passes=False, use_tc_tiling_on_sc=False),
        out_type=jax.ShapeDtypeStruct((48 * NP,), jnp.float32),
        scratch_types=[
            pltpu.VMEM((NP,), jnp.float32),
            pltpu.VMEM((NP,), jnp.float32),
            pltpu.VMEM((NP,), jnp.float32),
            pltpu.VMEM((NP,), jnp.float32),
            pltpu.VMEM((2000,), jnp.int32),
            pltpu.VMEM((2000,), jnp.int32),
        ],
    )
    def k(dst_h, src_h, eig1_h, out_h, *rest):
        _sc_pass0_body(dst_h, src_h, eig1_h, out_h, *rest)

    return k


def _sc_pass0(dst, src, eig1):
    return _build_pass0()(dst, src, eig1)


def _sc_edge_body(A2_h, C2_h, src_h, dst_h, eig1_h, rabs_h,
                  s0_h, s1_h, s2_h, smax_h,
                  eig1_v, rabs_v, maxacc, dstb, srcb,
                  dstl, srcl, eidl, avl, dxl,
                  bufA, bufC, bufAv, bufDx,
                  acc0, acc1, acc2, sem, sem2):
    c = lax.axis_index("c")
    s = lax.axis_index("s")

    pltpu.sync_copy(eig1_h, eig1_v)
    pltpu.sync_copy(rabs_h, rabs_v)

    def _flush(fill):
        pltpu.async_copy(A2_h.at[srcl], bufA, sem).wait()
        pltpu.async_copy(C2_h.at[eidl], bufC, sem).wait()

        def _r(r, _):
            rs = jnp.full((16,), r, jnp.int32)
            loc = plsc.load_gather(dstl, [rs])[0] - s * R2
            avr = plsc.load_gather(avl, [rs])
            dxr = plsc.load_gather(dxl, [rs])
            for j in range(4):
                sl = pl.ds(j * 16, 16)
                mj = bufA[r, sl] + bufC[r, sl]
                bufA[r, sl] = mj
                bufAv[r, sl] = mj * avr
                bufDx[r, sl] = mj * dxr
                maxacc[loc, sl] = jnp.maximum(maxacc[loc, sl], mj)
            return 0
        lax.fori_loop(0, fill, _r, 0)

        pltpu.sync_copy(bufA, acc0.at[dstl], add=True)
        pltpu.sync_copy(bufAv, acc1.at[dstl], add=True)
        pltpu.sync_copy(bufDx, acc2.at[dstl], add=True)
        for q in range(KB // 16):
            dstl[pl.ds(q * 16, 16)] = _splat_i32(DUMP)

    for p in range(NPHASE):
        plo = p * NPH
        tlo = plo + s * R2
        thi = tlo + R2

        def _initm(r, _):
            for j in range(4):
                maxacc[r, pl.ds(j * 16, 16)] = jnp.full((16,), NEG,
                                                        jnp.float32)
            return 0
        lax.fori_loop(0, R2, _initm, 0)

        def _zb(r, _):
            for j in range(4):
                bufA[r, pl.ds(j * 16, 16)] = jnp.zeros((16,), jnp.float32)
            return 0
        lax.fori_loop(0, KB, _zb, 0)

        rpt = ACC_ROWS // 16
        zlo = s * rpt
        for acc in (acc0, acc1, acc2):
            for q in range(rpt // KB):
                pltpu.sync_copy(bufA, acc.at[pl.ds(zlo + q * KB, KB)])
            if rpt % KB:
                pltpu.sync_copy(bufA.at[pl.ds(0, rpt % KB)],
                                acc.at[pl.ds(zlo + (rpt // KB) * KB,
                                             rpt % KB)])

        for q in range(KB // 16):
            dstl[pl.ds(q * 16, 16)] = _splat_i32(DUMP)
            srcl[pl.ds(q * 16, 16)] = _splat_i32(0)
            eidl[pl.ds(q * 16, 16)] = _splat_i32(0)

        plsc.subcore_barrier()

        def _chunk(k, fill):
            b = 0
            pltpu.sync_copy(dst_h.at[pl.ds(k * CH, CH)],
                            dstb.at[pl.ds(0, CH)])
            pltpu.sync_copy(src_h.at[pl.ds(k * CH, CH)],
                            srcb.at[pl.ds(0, CH)])

            def _grp(g, fill):
                d = dstb[pl.ds(b + g * 16, 16)]
                msk = (d >= tlo) & (d < thi)
                cnt = jnp.reshape(plsc.all_reduce_population_count(msk),
                                  (-1,))[0]

                def _do(f):
                    sv = srcb[pl.ds(b + g * 16, 16)]
                    e1d = plsc.load_gather(eig1_v, [d])
                    e1s = plsc.load_gather(eig1_v, [sv])
                    dw = e1d - e1s
                    ab = plsc.load_gather(rabs_v, [d]) + 1e-30
                    av = jnp.abs(dw) / ab
                    dx = dw / ab
                    eid = k * CH + g * 16 + lax.iota(jnp.int32, 16)
                    fs = pl.ds(f, 16)
                    plsc.store_compressed(dstl.at[fs], d - plo, mask=msk)
                    plsc.store_compressed(srcl.at[fs], sv * 2 + c, mask=msk)
                    plsc.store_compressed(eidl.at[fs], eid * 2 + c, mask=msk)
                    plsc.store_compressed(avl.at[fs], av, mask=msk)
                    plsc.store_compressed(dxl.at[fs], dx, mask=msk)
                    return f + cnt

                fill = lax.cond(cnt > 0, _do, lambda f: f, fill)

                def _fl(f):
                    _flush(f)
                    return 0
                return lax.cond(fill > KB - 16, _fl, lambda f: f, fill)

            return lax.fori_loop(0, CH // 16, _grp, fill)

        fill = lax.fori_loop(0, NCH, _chunk, 0)

        def _fl2(f):
            _flush(f)
            return 0
        lax.cond(fill > 0, _fl2, lambda f: f, fill)

        plsc.subcore_barrier()

        obase = c * NP3 + plo + s * R2
        rem = R2 % KB
        for acc, out in ((acc0, s0_h), (acc1, s1_h), (acc2, s2_h)):
            for q in range(R2 // KB):
                pltpu.sync_copy(acc.at[pl.ds(s * R2 + q * KB, KB)], bufC)
                pltpu.sync_copy(bufC, out.at[pl.ds(obase + q * KB, KB)])
            if rem:
                pltpu.sync_copy(acc.at[pl.ds(s * R2 + (R2 // KB) * KB, rem)],
                                bufC.at[pl.ds(0, rem)])
                pltpu.sync_copy(bufC.at[pl.ds(0, rem)],
                                out.at[pl.ds(obase + (R2 // KB) * KB, rem)])
        pltpu.sync_copy(maxacc, smax_h.at[pl.ds(obase, R2)])

        plsc.subcore_barrier()


_sc_out4 = tuple(jax.ShapeDtypeStruct((2 * NP3, HH), jnp.float32)
                 for _ in range(4))


@functools.cache
def _build_edge():
    mesh = plsc.VectorSubcoreMesh(core_axis_name="c", subcore_axis_name="s")

    @functools.partial(
        pl.kernel, mesh=mesh,
        compiler_params=pltpu.CompilerParams(needs_layout_passes=False, use_tc_tiling_on_sc=False),
        out_type=_sc_out4,
        scratch_types=[
        pltpu.VMEM((NP,), jnp.float32),
        pltpu.VMEM((NP,), jnp.float32),
        pltpu.VMEM((R2, HH), jnp.float32),
        pltpu.VMEM((CH,), jnp.int32),
        pltpu.VMEM((CH,), jnp.int32),
        pltpu.VMEM((KB,), jnp.int32),
        pltpu.VMEM((KB,), jnp.int32),
        pltpu.VMEM((KB,), jnp.int32),
        pltpu.VMEM((KB,), jnp.float32),
        pltpu.VMEM((KB,), jnp.float32),
        pltpu.VMEM((KB, HH), jnp.float32),
        pltpu.VMEM((KB, HH), jnp.float32),
        pltpu.VMEM((KB, HH), jnp.float32),
        pltpu.VMEM((KB, HH), jnp.float32),
            pltpu.VMEM_SHARED((ACC_ROWS, HH), jnp.float32),
            pltpu.VMEM_SHARED((ACC_ROWS, HH), jnp.float32),
            pltpu.VMEM_SHARED((ACC_ROWS, HH), jnp.float32),
            pltpu.SemaphoreType.DMA,
            pltpu.SemaphoreType.DMA,
        ],
    )
    def k(A2_h, C2_h, src_h, dst_h, eig1_h, rabs_h, s0, s1, s2, sm, *rest):
        _sc_edge_body(A2_h, C2_h, src_h, dst_h, eig1_h, rabs_h, s0, s1, s2,
                      sm, *rest)

    return k


def _sc_edge(A2, C2, src, dst, eig1, rabs):
    return _build_edge()(A2, C2, src, dst, eig1, rabs)


BR = 1000


def _ln(x, g, b):
    m = jnp.mean(x, axis=-1, keepdims=True)
    v = jnp.mean((x - m) ** 2, axis=-1, keepdims=True)
    return (x - m) * lax.rsqrt(v + 1e-5) * g + b


def _mm(x, w):
    return lax.dot_general(x, w, (((1,), (1,)), ((), ())),
                           preferred_element_type=jnp.float32)


def _tk_stats_body(x_r, o_r):
    x = x_r[...]
    rows = [jnp.sum(x[a * 16:(a + 1) * 16], axis=0, keepdims=True)
            for a in range(3)]
    o_r[...] = jnp.concatenate(rows + [jnp.zeros((5, x.shape[1]),
                                                 jnp.float32)], axis=0)


def _tk_stats(x):
    return pl.pallas_call(
        _tk_stats_body,
        grid=(NP // 1024,),
        in_specs=[pl.BlockSpec((48, 1024), lambda i: (0, i))],
        out_specs=pl.BlockSpec((8, 1024), lambda i: (0, i)),
        out_shape=jax.ShapeDtypeStruct((8, NP), jnp.float32),
    )(x)


def _tk_in_body(x_r, w_r, b_r, g_r, bb_r, o_r):
    h = _mm(x_r[...], w_r[...]) + b_r[...]
    o_r[...] = jnp.maximum(_ln(h, g_r[...], bb_r[...]), 0.0)


def _tk_in(x, w, b, g, bb):
    return pl.pallas_call(
        _tk_in_body,
        grid=(N // BR,),
        in_specs=[
            pl.BlockSpec((BR, H), lambda i: (i, 0)),
            pl.BlockSpec((H, H), lambda i: (0, 0)),
            pl.BlockSpec((1, H), lambda i: (0, 0)),
            pl.BlockSpec((1, H), lambda i: (0, 0)),
            pl.BlockSpec((1, H), lambda i: (0, 0)),
        ],
        out_specs=pl.BlockSpec((BR, H), lambda i: (i, 0)),
        out_shape=jax.ShapeDtypeStruct((N, H), jnp.float32),
    )(x, w, b, g, bb)


def _tk_ab_body(h_r, ws_r, wd_r, a_r, b_r):
    a_r[...] = _mm(h_r[...], ws_r[...])
    b_r[...] = _mm(h_r[...], wd_r[...])


def _tk_ab(h, ws, wd):
    return pl.pallas_call(
        _tk_ab_body,
        grid=(N // BR,),
        in_specs=[
            pl.BlockSpec((BR, H), lambda i: (i, 0)),
            pl.BlockSpec((H, H), lambda i: (0, 0)),
            pl.BlockSpec((H, H), lambda i: (0, 0)),
        ],
        out_specs=[
            pl.BlockSpec((BR, H), lambda i: (i, 0)),
            pl.BlockSpec((BR, H), lambda i: (i, 0)),
        ],
        out_shape=[jax.ShapeDtypeStruct((N, H), jnp.float32),
                   jax.ShapeDtypeStruct((N, H), jnp.float32)],
    )(h, ws, wd)


ER = 2000


def _tk_c_body(ef_r, w0_r, b0_r, w1_r, b1_r, c0_r, c1_r):
    ef = ef_r[...]
    c0_r[...] = _mm(ef, w0_r[...]) + b0_r[...]
    c1_r[...] = _mm(ef, w1_r[...]) + b1_r[...]


def _tk_c(ef, w0, b0, w1, b1):
    return pl.pallas_call(
        _tk_c_body,
        grid=(E // ER,),
        in_specs=[
            pl.BlockSpec((ER, 16), lambda i: (i, 0)),
            pl.BlockSpec((H, 16), lambda i: (0, 0)),
            pl.BlockSpec((1, H), lambda i: (0, 0)),
            pl.BlockSpec((H, 16), lambda i: (0, 0)),
            pl.BlockSpec((1, H), lambda i: (0, 0)),
        ],
        out_specs=[
            pl.BlockSpec((ER, H), lambda i: (i, 0)),
            pl.BlockSpec((ER, H), lambda i: (i, 0)),
        ],
        out_shape=[jax.ShapeDtypeStruct((E, H), jnp.float32),
                   jax.ShapeDtypeStruct((E, H), jnp.float32)],
    )(ef, w0, b0, w1, b1)


def _tk_tower_body(h_r, b_r, s0_r, s1_r, s2_r, sm_r, st_r, wu_r, bu_r, o_r):
    h = h_r[...]
    B = b_r[...]
    s0 = jnp.concatenate([s0_r[0], s0_r[1]], axis=-1)
    s1 = jnp.concatenate([s1_r[0], s1_r[1]], axis=-1)
    s2 = jnp.concatenate([s2_r[0], s2_r[1]], axis=-1)
    smx = jnp.concatenate([sm_r[0], sm_r[1]], axis=-1)
    st = st_r[...]
    deg = st[:, 0:1]
    rabs = st[:, 1:2]
    rdw = st[:, 2:3]
    degs = jnp.maximum(deg, 1.0)
    absf = rabs + 1e-30
    sumav = rabs / absf
    sumdx = rdw / absf
    s_sum = s0 + deg * B
    s_mean = s_sum / degs
    s_max = jnp.where(deg > 0.0, smx + B, 0.0)
    s_av = s1 + sumav * B
    s_dx = jnp.abs(s2 + sumdx * B - h * sumdx)
    logd = jnp.log(degs + 1.0)
    amp = logd / DELTA
    att = DELTA / logd
    wu = wu_r[...]
    aggs = (s_mean, s_max, s_sum, s_av, s_dx)
    p0 = _mm(h, wu[:, 0:H])
    for a in range(5):
        p0 = p0 + _mm(aggs[a], wu[:, (1 + a) * H:(2 + a) * H])
    p1 = _mm(aggs[0], wu[:, 6 * H:7 * H])
    p2 = _mm(aggs[0], wu[:, 11 * H:12 * H])
    for a in range(1, 5):
        p1 = p1 + _mm(aggs[a], wu[:, (6 + a) * H:(7 + a) * H])
        p2 = p2 + _mm(aggs[a], wu[:, (11 + a) * H:(12 + a) * H])
    o_r[...] = p0 + amp * p1 + att * p2 + bu_r[...]


def _tk_tower(h, B, s0, s1, s2, sm, st, wu, bu):
    half = pl.BlockSpec((2, BR, HH), lambda i: (0, i, 0))
    return pl.pallas_call(
        _tk_tower_body,
        grid=(N // BR,),
        in_specs=[
            pl.BlockSpec((BR, H), lambda i: (i, 0)),
            pl.BlockSpec((BR, H), lambda i: (i, 0)),
            half, half, half, half,
            pl.BlockSpec((BR, 8), lambda i: (i, 0)),
            pl.BlockSpec((H, 16 * H), lambda i: (0, 0)),
            pl.BlockSpec((1, H), lambda i: (0, 0)),
        ],
        out_specs=pl.BlockSpec((BR, H), lambda i: (i, 0)),
        out_shape=jax.ShapeDtypeStruct((N, H), jnp.float32),
    )(h, B, s0, s1, s2, sm, st, wu, bu)


def _tk_post_body(tw_r, h_r, wm_r, bm_r, bg_r, bb_r, lg_r, lb_r, rs_r, o_r,
                  stats):
    i = pl.program_id(0)
    phase = i // (N // BR)

    @pl.when(phase == 0)
    def _p0():
        @pl.when(i == 0)
        def _z():
            stats[...] = jnp.zeros((8, H), jnp.float32)
        tw = tw_r[...]
        stats[0:1, :] = stats[0:1, :] + jnp.sum(tw, axis=0, keepdims=True)
        stats[1:2, :] = stats[1:2, :] + jnp.sum(tw * tw, axis=0, keepdims=True)

    @pl.when(phase == 1)
    def _p1():
        tw = tw_r[...]
        h = h_r[...]
        mu = stats[0:1, :] / N
        var = stats[1:2, :] / N - mu * mu
        t = (tw - mu) * lax.rsqrt(var + 1e-5) * bg_r[...] + bb_r[...]
        mixed = _mm(t, wm_r[...]) + bm_r[...]
        mixed = jnp.where(mixed > 0.0, mixed, 0.01 * mixed)
        conv = mixed + h
        hn = jnp.maximum(_ln(conv, lg_r[...], lb_r[...]), 0.0)
        gate = 1.0 / (1.0 + jnp.exp(-rs_r[...]))
        o_r[...] = hn + gate * h


def _tk_post(tw, h, wm, bm, bg, bb, lg, lb, rs):
    nb = N // BR
    return pl.pallas_call(
        _tk_post_body,
        grid=(2 * nb,),
        in_specs=[
            pl.BlockSpec((BR, H), lambda i: (i % nb, 0)),
            pl.BlockSpec((BR, H), lambda i: (i % nb, 0)),
            pl.BlockSpec((H, H), lambda i: (0, 0)),
            pl.BlockSpec((1, H), lambda i: (0, 0)),
            pl.BlockSpec((1, H), lambda i: (0, 0)),
            pl.BlockSpec((1, H), lambda i: (0, 0)),
            pl.BlockSpec((1, H), lambda i: (0, 0)),
            pl.BlockSpec((1, H), lambda i: (0, 0)),
            pl.BlockSpec((1, H), lambda i: (0, 0)),
        ],
        out_specs=pl.BlockSpec((BR, H), lambda i: (i % nb, 0)),
        out_shape=jax.ShapeDtypeStruct((N, H), jnp.float32),
        scratch_shapes=[pltpu.VMEM((8, H), jnp.float32)],
    )(tw, h, wm, bm, bg, bb, lg, lb, rs)


def _tk_out_body(h_r, w1_r, b1_r, g_r, bb_r, w2_r, b2_r, o_r):
    t = _mm(h_r[...], w1_r[...]) + b1_r[...]
    t = jnp.maximum(_ln(t, g_r[...], bb_r[...]), 0.0)
    o_r[...] = _mm(t, w2_r[...]) + b2_r[...]


def _tk_out(h, w1, b1, g, bb, w2, b2):
    return pl.pallas_call(
        _tk_out_body,
        grid=(N // BR,),
        in_specs=[
            pl.BlockSpec((BR, H), lambda i: (i, 0)),
            pl.BlockSpec((H, H), lambda i: (0, 0)),
            pl.BlockSpec((1, H), lambda i: (0, 0)),
            pl.BlockSpec((1, H), lambda i: (0, 0)),
            pl.BlockSpec((1, H), lambda i: (0, 0)),
            pl.BlockSpec((H, H), lambda i: (0, 0)),
            pl.BlockSpec((1, H), lambda i: (0, 0)),
        ],
        out_specs=pl.BlockSpec((BR, H), lambda i: (i, 0)),
        out_shape=jax.ShapeDtypeStruct((N, H), jnp.float32),
    )(h, w1, b1, g, bb, w2, b2)


def kernel(node_features, edge_index, edge_feat, eig_vec, W_in, b_in, ln_in_g,
           ln_in_b, W_M, b_M, W_U, b_U, bn_g, bn_b, W_mix, b_mix, ln_int_g,
           ln_int_b, res_scale, W_o1, b_o1, ln_o_g, ln_o_b, W_o2, b_o2):
    src = edge_index[0]
    dst = edge_index[1]
    eig1 = jnp.zeros((NP,), jnp.float32).at[:N].set(eig_vec[:, 1])

    r2 = lambda v: v.reshape(1, H)
    h = _tk_in(node_features, W_in, r2(b_in), r2(ln_in_g), r2(ln_in_b))

    parts = _sc_pass0(dst, src, eig1).reshape(48, NP)
    stats3 = _tk_stats(parts)
    st = jnp.zeros((N, 8), jnp.float32).at[:, 0:3].set(stats3[:3, :N].T)

    C0, C1 = _tk_c(edge_feat, W_M[0][:, 2 * H:], r2(b_M[0]),
                   W_M[1][:, 2 * H:], r2(b_M[1]))
    Cs = (C0, C1)

    for l in range(2):
        A, B = _tk_ab(h, W_M[l][:, :H], W_M[l][:, H:2 * H])
        A2 = A.reshape(2 * N, HH)
        C2 = Cs[l].reshape(2 * E, HH)
        s0, s1, s2, sm = _sc_edge(A2, C2, src, dst, eig1, stats3[1])
        r3 = lambda x: x.reshape(2, NP3, HH)[:, :N]
        tower = _tk_tower(h, B, r3(s0), r3(s1), r3(s2), r3(sm),
                          st, W_U[l], r2(b_U[l]))
        h = _tk_post(tower, h, W_mix[l], r2(b_mix[l]), r2(bn_g[l]),
                     r2(bn_b[l]), r2(ln_int_g[l]), r2(ln_int_b[l]),
                     jnp.full((1, H), res_scale[l], jnp.float32))

    return _tk_out(h, W_o1, r2(b_o1), r2(ln_o_g), r2(ln_o_b), W_o2, r2(b_o2))


# speedup vs baseline: 2.0119x; 1.0027x over previous
import functools
import jax
import jax.numpy as jnp
from jax import lax
from jax.experimental import pallas as pl
from jax.experimental.pallas import tpu as pltpu
from jax.experimental.pallas import tpu_sc as plsc

N = 10000
E = 160000
H = 128
HH = 64
NP = 10240
NPH = 3456
NPHASE = 3
NP3 = NPHASE * NPH
R2 = NPH // 16
ACC_ROWS = NPH + 16
DUMP = NPH
KB = 128
CH = 8000
NCH = E // CH
NEG = -3.0e38
DELTA = 1.0

def _splat_i32(v):
    return jnp.full((16,), v, jnp.int32)


def _sc_pass0_body(dst_h, src_h, eig1_h, out_h, eig1_v, accd, acca, accw,
                   dstb, srcb):
    c = lax.axis_index("c")
    s = lax.axis_index("s")

    @pl.when(c == 0)
    def _work():
        pltpu.sync_copy(eig1_h, eig1_v)
        l0 = jnp.where(lax.iota(jnp.int32, 16) == 0, 1.0, 0.0)

        def _z(r, _):
            z = jnp.zeros((16,), jnp.float32)
            accd[pl.ds(r * 16, 16)] = z
            acca[pl.ds(r * 16, 16)] = z
            accw[pl.ds(r * 16, 16)] = z
            return 0
        lax.fori_loop(0, NP // 16, _z, 0)

        base = s * (E // 16)

        def _chunk(k, _):
            pltpu.sync_copy(dst_h.at[pl.ds(base + k * 2000, 2000)], dstb)
            pltpu.sync_copy(src_h.at[pl.ds(base + k * 2000, 2000)], srcb)

            def _grp(g, _):
                d = dstb[pl.ds(g * 16, 16)]
                sv = srcb[pl.ds(g * 16, 16)]
                e1d = plsc.load_gather(eig1_v, [d])
                e1s = plsc.load_gather(eig1_v, [sv])
                dw = e1d - e1s
                adw = jnp.abs(dw)
                for j in range(16):
                    dj = d[j]
                    sl = pl.ds(dj, 16)
                    accd[sl] = accd[sl] + l0
                    acca[sl] = acca[sl] + l0 * adw[j]
                    accw[sl] = accw[sl] + l0 * dw[j]
                return 0
            lax.fori_loop(0, 2000 // 16, _grp, 0)
            return 0
        lax.fori_loop(0, (E // 16) // 2000, _chunk, 0)

        for a, acc in enumerate((accd, acca, accw)):
            pltpu.sync_copy(acc, out_h.at[pl.ds((a * 16 + s) * NP, NP)])


@functools.cache
def _build_pass0():
    mesh = plsc.VectorSubcoreMesh(core_axis_name="c", subcore_axis_name="s")

    @functools.partial(
        pl.kernel, mesh=mesh,
        compiler_params=pltpu.CompilerParams(needs_layout_passes=False, use_tc_tiling_on_sc=False),
        out_type=jax.ShapeDtypeStruct((48 * NP,), jnp.float32),
        scratch_types=[
            pltpu.VMEM((NP,), jnp.float32),
            pltpu.VMEM((NP,), jnp.float32),
            pltpu.VMEM((NP,), jnp.float32),
            pltpu.VMEM((NP,), jnp.float32),
            pltpu.VMEM((2000,), jnp.int32),
            pltpu.VMEM((2000,), jnp.int32),
        ],
    )
    def k(dst_h, src_h, eig1_h, out_h, *rest):
        _sc_pass0_body(dst_h, src_h, eig1_h, out_h, *rest)

    return k


def _sc_pass0(dst, src, eig1):
    return _build_pass0()(dst, src, eig1)


def _sc_edge_body(A2_h, C2_h, src_h, dst_h, eig1_h, rabs_h,
                  s0_h, s1_h, s2_h, smax_h,
                  eig1_v, rabs_v, maxacc, dstb, srcb,
                  dstl, srcl, eidl, avl, dxl,
                  bufA, bufC, bufAv, bufDx,
                  acc0, acc1, acc2, sem, sem2):
    c = lax.axis_index("c")
    s = lax.axis_index("s")

    pltpu.sync_copy(eig1_h, eig1_v)
    pltpu.sync_copy(rabs_h, rabs_v)

    def _flush(fill):
        del fill
        pltpu.async_copy(A2_h.at[srcl], bufA, sem).wait()
        pltpu.async_copy(C2_h.at[eidl], bufC, sem).wait()

        def _q(q, _):
            dstv = dstl[pl.ds(q * 16, 16)]
            avv = avl[pl.ds(q * 16, 16)]
            dxv = dxl[pl.ds(q * 16, 16)]
            for j in range(16):
                r = q * 16 + j
                loc = jnp.minimum(dstv[j] - s * R2, R2)
                avj = avv[j]
                dxj = dxv[j]
                for t in range(4):
                    sl = pl.ds(t * 16, 16)
                    mj = bufA[r, sl] + bufC[r, sl]
                    bufA[r, sl] = mj
                    bufAv[r, sl] = mj * avj
                    bufDx[r, sl] = mj * dxj
                    maxacc[loc, sl] = jnp.maximum(maxacc[loc, sl], mj)
            return 0
        lax.fori_loop(0, KB // 16, _q, 0)

        pltpu.sync_copy(bufA, acc0.at[dstl], add=True)
        pltpu.sync_copy(bufAv, acc1.at[dstl], add=True)
        pltpu.sync_copy(bufDx, acc2.at[dstl], add=True)
        for q in range(KB // 16):
            dstl[pl.ds(q * 16, 16)] = _splat_i32(DUMP)

    for p in range(NPHASE):
        plo = p * NPH
        tlo = plo + s * R2
        thi = tlo + R2

        def _initm(r, _):
            for j in range(4):
                maxacc[r, pl.ds(j * 16, 16)] = jnp.full((16,), NEG,
                                                        jnp.float32)
            return 0
        lax.fori_loop(0, R2 + 1, _initm, 0)

        def _zb(r, _):
            for j in range(4):
                bufA[r, pl.ds(j * 16, 16)] = jnp.zeros((16,), jnp.float32)
            return 0
        lax.fori_loop(0, KB, _zb, 0)

        rpt = ACC_ROWS // 16
        zlo = s * rpt
        for acc in (acc0, acc1, acc2):
            for q in range(rpt // KB):
                pltpu.sync_copy(bufA, acc.at[pl.ds(zlo + q * KB, KB)])
            if rpt % KB:
                pltpu.sync_copy(bufA.at[pl.ds(0, rpt % KB)],
                                acc.at[pl.ds(zlo + (rpt // KB) * KB,
                                             rpt % KB)])

        for q in range(KB // 16):
            dstl[pl.ds(q * 16, 16)] = _splat_i32(DUMP)
            srcl[pl.ds(q * 16, 16)] = _splat_i32(0)
            eidl[pl.ds(q * 16, 16)] = _splat_i32(0)

        plsc.subcore_barrier()

        def _chunk(k, fill):
            b = 0
            pltpu.sync_copy(dst_h.at[pl.ds(k * CH, CH)],
                            dstb.at[pl.ds(0, CH)])
            pltpu.sync_copy(src_h.at[pl.ds(k * CH, CH)],
                            srcb.at[pl.ds(0, CH)])

            def _grp(g, fill):
                d = dstb[pl.ds(b + g * 16, 16)]
                msk = (d >= tlo) & (d < thi)
                cnt = jnp.reshape(plsc.all_reduce_population_count(msk),
                                  (-1,))[0]

                def _do(f):
                    sv = srcb[pl.ds(b + g * 16, 16)]
                    e1d = plsc.load_gather(eig1_v, [d])
                    e1s = plsc.load_gather(eig1_v, [sv])
                    dw = e1d - e1s
                    ab = plsc.load_gather(rabs_v, [d]) + 1e-30
                    av = jnp.abs(dw) / ab
                    dx = dw / ab
                    eid = k * CH + g * 16 + lax.iota(jnp.int32, 16)
                    fs = pl.ds(f, 16)
                    plsc.store_compressed(dstl.at[fs], d - plo, mask=msk)
                    plsc.store_compressed(srcl.at[fs], sv * 2 + c, mask=msk)
                    plsc.store_compressed(eidl.at[fs], eid * 2 + c, mask=msk)
                    plsc.store_compressed(avl.at[fs], av, mask=msk)
                    plsc.store_compressed(dxl.at[fs], dx, mask=msk)
                    return f + cnt

                fill = lax.cond(cnt > 0, _do, lambda f: f, fill)

                def _fl(f):
                    _flush(f)
                    return 0
                return lax.cond(fill > KB - 16, _fl, lambda f: f, fill)

            return lax.fori_loop(0, CH // 16, _grp, fill)

        fill = lax.fori_loop(0, NCH, _chunk, 0)

        def _fl2(f):
            _flush(f)
            return 0
        lax.cond(fill > 0, _fl2, lambda f: f, fill)

        plsc.subcore_barrier()

        obase = c * NP3 + plo + s * R2
        rem = R2 % KB
        for acc, out in ((acc0, s0_h), (acc1, s1_h), (acc2, s2_h)):
            for q in range(R2 // KB):
                pltpu.sync_copy(acc.at[pl.ds(s * R2 + q * KB, KB)], bufC)
                pltpu.sync_copy(bufC, out.at[pl.ds(obase + q * KB, KB)])
            if rem:
                pltpu.sync_copy(acc.at[pl.ds(s * R2 + (R2 // KB) * KB, rem)],
                                bufC.at[pl.ds(0, rem)])
                pltpu.sync_copy(bufC.at[pl.ds(0, rem)],
                                out.at[pl.ds(obase + (R2 // KB) * KB, rem)])
        pltpu.sync_copy(maxacc.at[pl.ds(0, R2)],
                        smax_h.at[pl.ds(obase, R2)])

        plsc.subcore_barrier()


_sc_out4 = tuple(jax.ShapeDtypeStruct((2 * NP3, HH), jnp.float32)
                 for _ in range(4))


@functools.cache
def _build_edge():
    mesh = plsc.VectorSubcoreMesh(core_axis_name="c", subcore_axis_name="s")

    @functools.partial(
        pl.kernel, mesh=mesh,
        compiler_params=pltpu.CompilerParams(needs_layout_passes=False, use_tc_tiling_on_sc=False),
        out_type=_sc_out4,
        scratch_types=[
        pltpu.VMEM((NP,), jnp.float32),
        pltpu.VMEM((NP,), jnp.float32),
        pltpu.VMEM((R2 + 1, HH), jnp.float32),
        pltpu.VMEM((CH,), jnp.int32),
        pltpu.VMEM((CH,), jnp.int32),
        pltpu.VMEM((KB,), jnp.int32),
        pltpu.VMEM((KB,), jnp.int32),
        pltpu.VMEM((KB,), jnp.int32),
        pltpu.VMEM((KB,), jnp.float32),
        pltpu.VMEM((KB,), jnp.float32),
        pltpu.VMEM((KB, HH), jnp.float32),
        pltpu.VMEM((KB, HH), jnp.float32),
        pltpu.VMEM((KB, HH), jnp.float32),
        pltpu.VMEM((KB, HH), jnp.float32),
            pltpu.VMEM_SHARED((ACC_ROWS, HH), jnp.float32),
            pltpu.VMEM_SHARED((ACC_ROWS, HH), jnp.float32),
            pltpu.VMEM_SHARED((ACC_ROWS, HH), jnp.float32),
            pltpu.SemaphoreType.DMA,
            pltpu.SemaphoreType.DMA,
        ],
    )
    def k(A2_h, C2_h, src_h, dst_h, eig1_h, rabs_h, s0, s1, s2, sm, *rest):
        _sc_edge_body(A2_h, C2_h, src_h, dst_h, eig1_h, rabs_h, s0, s1, s2,
                      sm, *rest)

    return k


def _sc_edge(A2, C2, src, dst, eig1, rabs):
    return _build_edge()(A2, C2, src, dst, eig1, rabs)


BR = 1000


def _ln(x, g, b):
    m = jnp.mean(x, axis=-1, keepdims=True)
    v = jnp.mean((x - m) ** 2, axis=-1, keepdims=True)
    return (x - m) * lax.rsqrt(v + 1e-5) * g + b


def _mm(x, w):
    return lax.dot_general(x, w, (((1,), (1,)), ((), ())),
                           preferred_element_type=jnp.float32)


def _tk_stats_body(x_r, o_r):
    x = x_r[...]
    rows = [jnp.sum(x[a * 16:(a + 1) * 16], axis=0, keepdims=True)
            for a in range(3)]
    o_r[...] = jnp.concatenate(rows + [jnp.zeros((5, x.shape[1]),
                                                 jnp.float32)], axis=0)


def _tk_stats(x):
    return pl.pallas_call(
        _tk_stats_body,
        grid=(NP // 1024,),
        in_specs=[pl.BlockSpec((48, 1024), lambda i: (0, i))],
        out_specs=pl.BlockSpec((8, 1024), lambda i: (0, i)),
        out_shape=jax.ShapeDtypeStruct((8, NP), jnp.float32),
    )(x)


def _tk_in_body(x_r, w_r, b_r, g_r, bb_r, o_r):
    h = _mm(x_r[...], w_r[...]) + b_r[...]
    o_r[...] = jnp.maximum(_ln(h, g_r[...], bb_r[...]), 0.0)


def _tk_in(x, w, b, g, bb):
    return pl.pallas_call(
        _tk_in_body,
        grid=(N // BR,),
        in_specs=[
            pl.BlockSpec((BR, H), lambda i: (i, 0)),
            pl.BlockSpec((H, H), lambda i: (0, 0)),
            pl.BlockSpec((1, H), lambda i: (0, 0)),
            pl.BlockSpec((1, H), lambda i: (0, 0)),
            pl.BlockSpec((1, H), lambda i: (0, 0)),
        ],
        out_specs=pl.BlockSpec((BR, H), lambda i: (i, 0)),
        out_shape=jax.ShapeDtypeStruct((N, H), jnp.float32),
    )(x, w, b, g, bb)


def _tk_ab_body(h_r, ws_r, wd_r, a_r, b_r):
    a_r[...] = _mm(h_r[...], ws_r[...])
    b_r[...] = _mm(h_r[...], wd_r[...])


def _tk_ab(h, ws, wd):
    return pl.pallas_call(
        _tk_ab_body,
        grid=(N // BR,),
        in_specs=[
            pl.BlockSpec((BR, H), lambda i: (i, 0)),
            pl.BlockSpec((H, H), lambda i: (0, 0)),
            pl.BlockSpec((H, H), lambda i: (0, 0)),
        ],
        out_specs=[
            pl.BlockSpec((BR, H), lambda i: (i, 0)),
            pl.BlockSpec((BR, H), lambda i: (i, 0)),
        ],
        out_shape=[jax.ShapeDtypeStruct((N, H), jnp.float32),
                   jax.ShapeDtypeStruct((N, H), jnp.float32)],
    )(h, ws, wd)


ER = 2000


def _tk_c_body(ef_r, w0_r, b0_r, w1_r, b1_r, c0_r, c1_r):
    ef = ef_r[...]
    c0_r[...] = _mm(ef, w0_r[...]) + b0_r[...]
    c1_r[...] = _mm(ef, w1_r[...]) + b1_r[...]


def _tk_c(ef, w0, b0, w1, b1):
    return pl.pallas_call(
        _tk_c_body,
        grid=(E // ER,),
        in_specs=[
            pl.BlockSpec((ER, 16), lambda i: (i, 0)),
            pl.BlockSpec((H, 16), lambda i: (0, 0)),
            pl.BlockSpec((1, H), lambda i: (0, 0)),
            pl.BlockSpec((H, 16), lambda i: (0, 0)),
            pl.BlockSpec((1, H), lambda i: (0, 0)),
        ],
        out_specs=[
            pl.BlockSpec((ER, H), lambda i: (i, 0)),
            pl.BlockSpec((ER, H), lambda i: (i, 0)),
        ],
        out_shape=[jax.ShapeDtypeStruct((E, H), jnp.float32),
                   jax.ShapeDtypeStruct((E, H), jnp.float32)],
    )(ef, w0, b0, w1, b1)


def _tk_tower_body(h_r, b_r, s0_r, s1_r, s2_r, sm_r, st_r, wu_r, bu_r, o_r):
    h = h_r[...]
    B = b_r[...]
    s0 = jnp.concatenate([s0_r[0], s0_r[1]], axis=-1)
    s1 = jnp.concatenate([s1_r[0], s1_r[1]], axis=-1)
    s2 = jnp.concatenate([s2_r[0], s2_r[1]], axis=-1)
    smx = jnp.concatenate([sm_r[0], sm_r[1]], axis=-1)
    st = st_r[...]
    deg = st[:, 0:1]
    rabs = st[:, 1:2]
    rdw = st[:, 2:3]
    degs = jnp.maximum(deg, 1.0)
    absf = rabs + 1e-30
    sumav = rabs / absf
    sumdx = rdw / absf
    s_sum = s0 + deg * B
    s_mean = s_sum / degs
    s_max = jnp.where(deg > 0.0, smx + B, 0.0)
    s_av = s1 + sumav * B
    s_dx = jnp.abs(s2 + sumdx * B - h * sumdx)
    logd = jnp.log(degs + 1.0)
    amp = logd / DELTA
    att = DELTA / logd
    wu = wu_r[...]
    aggs = (s_mean, s_max, s_sum, s_av, s_dx)
    p0 = _mm(h, wu[:, 0:H])
    for a in range(5):
        p0 = p0 + _mm(aggs[a], wu[:, (1 + a) * H:(2 + a) * H])
    p1 = _mm(aggs[0], wu[:, 6 * H:7 * H])
    p2 = _mm(aggs[0], wu[:, 11 * H:12 * H])
    for a in range(1, 5):
        p1 = p1 + _mm(aggs[a], wu[:, (6 + a) * H:(7 + a) * H])
        p2 = p2 + _mm(aggs[a], wu[:, (11 + a) * H:(12 + a) * H])
    o_r[...] = p0 + amp * p1 + att * p2 + bu_r[...]


def _tk_tower(h, B, s0, s1, s2, sm, st, wu, bu):
    half = pl.BlockSpec((2, BR, HH), lambda i: (0, i, 0))
    return pl.pallas_call(
        _tk_tower_body,
        grid=(N // BR,),
        in_specs=[
            pl.BlockSpec((BR, H), lambda i: (i, 0)),
            pl.BlockSpec((BR, H), lambda i: (i, 0)),
            half, half, half, half,
            pl.BlockSpec((BR, 8), lambda i: (i, 0)),
            pl.BlockSpec((H, 16 * H), lambda i: (0, 0)),
            pl.BlockSpec((1, H), lambda i: (0, 0)),
        ],
        out_specs=pl.BlockSpec((BR, H), lambda i: (i, 0)),
        out_shape=jax.ShapeDtypeStruct((N, H), jnp.float32),
    )(h, B, s0, s1, s2, sm, st, wu, bu)


def _tk_post_body(tw_r, h_r, wm_r, bm_r, bg_r, bb_r, lg_r, lb_r, rs_r, o_r,
                  stats):
    i = pl.program_id(0)
    phase = i // (N // BR)

    @pl.when(phase == 0)
    def _p0():
        @pl.when(i == 0)
        def _z():
            stats[...] = jnp.zeros((8, H), jnp.float32)
        tw = tw_r[...]
        stats[0:1, :] = stats[0:1, :] + jnp.sum(tw, axis=0, keepdims=True)
        stats[1:2, :] = stats[1:2, :] + jnp.sum(tw * tw, axis=0, keepdims=True)

    @pl.when(phase == 1)
    def _p1():
        tw = tw_r[...]
        h = h_r[...]
        mu = stats[0:1, :] / N
        var = stats[1:2, :] / N - mu * mu
        t = (tw - mu) * lax.rsqrt(var + 1e-5) * bg_r[...] + bb_r[...]
        mixed = _mm(t, wm_r[...]) + bm_r[...]
        mixed = jnp.where(mixed > 0.0, mixed, 0.01 * mixed)
        conv = mixed + h
        hn = jnp.maximum(_ln(conv, lg_r[...], lb_r[...]), 0.0)
        gate = 1.0 / (1.0 + jnp.exp(-rs_r[...]))
        o_r[...] = hn + gate * h


def _tk_post(tw, h, wm, bm, bg, bb, lg, lb, rs):
    nb = N // BR
    return pl.pallas_call(
        _tk_post_body,
        grid=(2 * nb,),
        in_specs=[
            pl.BlockSpec((BR, H), lambda i: (i % nb, 0)),
            pl.BlockSpec((BR, H), lambda i: (i % nb, 0)),
            pl.BlockSpec((H, H), lambda i: (0, 0)),
            pl.BlockSpec((1, H), lambda i: (0, 0)),
            pl.BlockSpec((1, H), lambda i: (0, 0)),
            pl.BlockSpec((1, H), lambda i: (0, 0)),
            pl.BlockSpec((1, H), lambda i: (0, 0)),
            pl.BlockSpec((1, H), lambda i: (0, 0)),
            pl.BlockSpec((1, H), lambda i: (0, 0)),
        ],
        out_specs=pl.BlockSpec((BR, H), lambda i: (i % nb, 0)),
        out_shape=jax.ShapeDtypeStruct((N, H), jnp.float32),
        scratch_shapes=[pltpu.VMEM((8, H), jnp.float32)],
    )(tw, h, wm, bm, bg, bb, lg, lb, rs)


def _tk_out_body(h_r, w1_r, b1_r, g_r, bb_r, w2_r, b2_r, o_r):
    t = _mm(h_r[...], w1_r[...]) + b1_r[...]
    t = jnp.maximum(_ln(t, g_r[...], bb_r[...]), 0.0)
    o_r[...] = _mm(t, w2_r[...]) + b2_r[...]


def _tk_out(h, w1, b1, g, bb, w2, b2):
    return pl.pallas_call(
        _tk_out_body,
        grid=(N // BR,),
        in_specs=[
            pl.BlockSpec((BR, H), lambda i: (i, 0)),
            pl.BlockSpec((H, H), lambda i: (0, 0)),
            pl.BlockSpec((1, H), lambda i: (0, 0)),
            pl.BlockSpec((1, H), lambda i: (0, 0)),
            pl.BlockSpec((1, H), lambda i: (0, 0)),
            pl.BlockSpec((H, H), lambda i: (0, 0)),
            pl.BlockSpec((1, H), lambda i: (0, 0)),
        ],
        out_specs=pl.BlockSpec((BR, H), lambda i: (i, 0)),
        out_shape=jax.ShapeDtypeStruct((N, H), jnp.float32),
    )(h, w1, b1, g, bb, w2, b2)


def kernel(node_features, edge_index, edge_feat, eig_vec, W_in, b_in, ln_in_g,
           ln_in_b, W_M, b_M, W_U, b_U, bn_g, bn_b, W_mix, b_mix, ln_int_g,
           ln_int_b, res_scale, W_o1, b_o1, ln_o_g, ln_o_b, W_o2, b_o2):
    src = edge_index[0]
    dst = edge_index[1]
    eig1 = jnp.zeros((NP,), jnp.float32).at[:N].set(eig_vec[:, 1])

    r2 = lambda v: v.reshape(1, H)
    h = _tk_in(node_features, W_in, r2(b_in), r2(ln_in_g), r2(ln_in_b))

    parts = _sc_pass0(dst, src, eig1).reshape(48, NP)
    stats3 = _tk_stats(parts)
    st = jnp.zeros((N, 8), jnp.float32).at[:, 0:3].set(stats3[:3, :N].T)

    C0, C1 = _tk_c(edge_feat, W_M[0][:, 2 * H:], r2(b_M[0]),
                   W_M[1][:, 2 * H:], r2(b_M[1]))
    Cs = (C0, C1)

    for l in range(2):
        A, B = _tk_ab(h, W_M[l][:, :H], W_M[l][:, H:2 * H])
        A2 = A.reshape(2 * N, HH)
        C2 = Cs[l].reshape(2 * E, HH)
        s0, s1, s2, sm = _sc_edge(A2, C2, src, dst, eig1, stats3[1])
        r3 = lambda x: x.reshape(2, NP3, HH)[:, :N]
        tower = _tk_tower(h, B, r3(s0), r3(s1), r3(s2), r3(sm),
                          st, W_U[l], r2(b_U[l]))
        h = _tk_post(tower, h, W_mix[l], r2(b_mix[l]), r2(bn_g[l]),
                     r2(bn_b[l]), r2(ln_int_g[l]), r2(ln_int_b[l]),
                     jnp.full((1, H), res_scale[l], jnp.float32))

    return _tk_out(h, W_o1, r2(b_o1), r2(ln_o_g), r2(ln_o_b), W_o2, r2(b_o2))


# speedup vs baseline: 2.1195x; 1.0535x over previous
import functools
import jax
import jax.numpy as jnp
from jax import lax
from jax.experimental import pallas as pl
from jax.experimental.pallas import tpu as pltpu
from jax.experimental.pallas import tpu_sc as plsc

N = 10000
E = 160000
H = 128
HH = 64
NP = 10240
NPH = 3456
NPHASE = 3
NP3 = NPHASE * NPH
R2 = NPH // 16
ACC_ROWS = NPH + 16
DUMP = NPH
KB = 128
CH = 8000
NCH = E // CH
NEG = -3.0e38
DELTA = 1.0

def _splat_i32(v):
    return jnp.full((16,), v, jnp.int32)


def _sc_pass0_body(dst_h, src_h, eig1_h, out_h, eig1_v, accd, acca, accw,
                   dstb, srcb):
    c = lax.axis_index("c")
    s = lax.axis_index("s")

    @pl.when(c == 0)
    def _work():
        pltpu.sync_copy(eig1_h, eig1_v)
        l0 = jnp.where(lax.iota(jnp.int32, 16) == 0, 1.0, 0.0)

        def _z(r, _):
            z = jnp.zeros((16,), jnp.float32)
            accd[pl.ds(r * 16, 16)] = z
            acca[pl.ds(r * 16, 16)] = z
            accw[pl.ds(r * 16, 16)] = z
            return 0
        lax.fori_loop(0, NP // 16, _z, 0)

        base = s * (E // 16)

        def _chunk(k, _):
            pltpu.sync_copy(dst_h.at[pl.ds(base + k * 2000, 2000)], dstb)
            pltpu.sync_copy(src_h.at[pl.ds(base + k * 2000, 2000)], srcb)

            def _grp(g, _):
                d = dstb[pl.ds(g * 16, 16)]
                sv = srcb[pl.ds(g * 16, 16)]
                e1d = plsc.load_gather(eig1_v, [d])
                e1s = plsc.load_gather(eig1_v, [sv])
                dw = e1d - e1s
                adw = jnp.abs(dw)
                for j in range(16):
                    dj = d[j]
                    sl = pl.ds(dj, 16)
                    accd[sl] = accd[sl] + l0
                    acca[sl] = acca[sl] + l0 * adw[j]
                    accw[sl] = accw[sl] + l0 * dw[j]
                return 0
            lax.fori_loop(0, 2000 // 16, _grp, 0)
            return 0
        lax.fori_loop(0, (E // 16) // 2000, _chunk, 0)

        for a, acc in enumerate((accd, acca, accw)):
            pltpu.sync_copy(acc, out_h.at[pl.ds((a * 16 + s) * NP, NP)])


@functools.cache
def _build_pass0():
    mesh = plsc.VectorSubcoreMesh(core_axis_name="c", subcore_axis_name="s")

    @functools.partial(
        pl.kernel, mesh=mesh,
        compiler_params=pltpu.CompilerParams(needs_layout_passes=False, use_tc_tiling_on_sc=False),
        out_type=jax.ShapeDtypeStruct((48 * NP,), jnp.float32),
        scratch_types=[
            pltpu.VMEM((NP,), jnp.float32),
            pltpu.VMEM((NP,), jnp.float32),
            pltpu.VMEM((NP,), jnp.float32),
            pltpu.VMEM((NP,), jnp.float32),
            pltpu.VMEM((2000,), jnp.int32),
            pltpu.VMEM((2000,), jnp.int32),
        ],
    )
    def k(dst_h, src_h, eig1_h, out_h, *rest):
        _sc_pass0_body(dst_h, src_h, eig1_h, out_h, *rest)

    return k


def _sc_pass0(dst, src, eig1):
    return _build_pass0()(dst, src, eig1)


def _sc_edge_body(A2_h, C2_h, src_h, dst_h, eig1_h, rabs_h,
                  s0_h, s1_h, s2_h, smax_h,
                  eig1_v, rabs_v, maxacc, dstb, srcb,
                  dstl, srcl, eidl, avl, dxl,
                  bufA, bufC, bufAv, bufDx,
                  acc0, acc1, acc2, sem, sem2, sem3):
    c = lax.axis_index("c")
    s = lax.axis_index("s")

    pltpu.sync_copy(eig1_h, eig1_v)
    pltpu.sync_copy(rabs_h, rabs_v)

    def _flush(fill):
        del fill
        ga = pltpu.async_copy(A2_h.at[srcl], bufA, sem)
        gc = pltpu.async_copy(C2_h.at[eidl], bufC, sem2)
        ga.wait()
        gc.wait()

        def _q(q, _):
            dstv = dstl[pl.ds(q * 16, 16)]
            avv = avl[pl.ds(q * 16, 16)]
            dxv = dxl[pl.ds(q * 16, 16)]
            for j in range(16):
                r = q * 16 + j
                loc = jnp.minimum(dstv[j] - s * R2, R2)
                avj = avv[j]
                dxj = dxv[j]
                for t in range(4):
                    sl = pl.ds(t * 16, 16)
                    mj = bufA[r, sl] + bufC[r, sl]
                    bufA[r, sl] = mj
                    bufAv[r, sl] = mj * avj
                    bufDx[r, sl] = mj * dxj
                    maxacc[loc, sl] = jnp.maximum(maxacc[loc, sl], mj)
            return 0
        lax.fori_loop(0, KB // 16, _q, 0)

        sa = pltpu.async_copy(bufA, acc0.at[dstl], sem, add=True)
        sb = pltpu.async_copy(bufAv, acc1.at[dstl], sem2, add=True)
        sc = pltpu.async_copy(bufDx, acc2.at[dstl], sem3, add=True)
        sa.wait()
        sb.wait()
        sc.wait()
        for q in range(KB // 16):
            dstl[pl.ds(q * 16, 16)] = _splat_i32(DUMP)

    for p in range(NPHASE):
        plo = p * NPH
        tlo = plo + s * R2
        thi = tlo + R2

        def _initm(r, _):
            for j in range(4):
                maxacc[r, pl.ds(j * 16, 16)] = jnp.full((16,), NEG,
                                                        jnp.float32)
            return 0
        lax.fori_loop(0, R2 + 1, _initm, 0)

        def _zb(r, _):
            for j in range(4):
                bufA[r, pl.ds(j * 16, 16)] = jnp.zeros((16,), jnp.float32)
            return 0
        lax.fori_loop(0, KB, _zb, 0)

        rpt = ACC_ROWS // 16
        zlo = s * rpt
        for acc in (acc0, acc1, acc2):
            for q in range(rpt // KB):
                pltpu.sync_copy(bufA, acc.at[pl.ds(zlo + q * KB, KB)])
            if rpt % KB:
                pltpu.sync_copy(bufA.at[pl.ds(0, rpt % KB)],
                                acc.at[pl.ds(zlo + (rpt // KB) * KB,
                                             rpt % KB)])

        for q in range(KB // 16):
            dstl[pl.ds(q * 16, 16)] = _splat_i32(DUMP)
            srcl[pl.ds(q * 16, 16)] = _splat_i32(0)
            eidl[pl.ds(q * 16, 16)] = _splat_i32(0)

        plsc.subcore_barrier()

        def _chunk(k, fill):
            b = 0
            pltpu.sync_copy(dst_h.at[pl.ds(k * CH, CH)],
                            dstb.at[pl.ds(0, CH)])
            pltpu.sync_copy(src_h.at[pl.ds(k * CH, CH)],
                            srcb.at[pl.ds(0, CH)])

            def _grp(g, fill):
                d = dstb[pl.ds(b + g * 16, 16)]
                msk = (d >= tlo) & (d < thi)
                cnt = jnp.reshape(plsc.all_reduce_population_count(msk),
                                  (-1,))[0]

                def _do(f):
                    sv = srcb[pl.ds(b + g * 16, 16)]
                    e1d = plsc.load_gather(eig1_v, [d])
                    e1s = plsc.load_gather(eig1_v, [sv])
                    dw = e1d - e1s
                    ab = plsc.load_gather(rabs_v, [d]) + 1e-30
                    av = jnp.abs(dw) / ab
                    dx = dw / ab
                    eid = k * CH + g * 16 + lax.iota(jnp.int32, 16)
                    fs = pl.ds(f, 16)
                    plsc.store_compressed(dstl.at[fs], d - plo, mask=msk)
                    plsc.store_compressed(srcl.at[fs], sv * 2 + c, mask=msk)
                    plsc.store_compressed(eidl.at[fs], eid * 2 + c, mask=msk)
                    plsc.store_compressed(avl.at[fs], av, mask=msk)
                    plsc.store_compressed(dxl.at[fs], dx, mask=msk)
                    return f + cnt

                fill = lax.cond(cnt > 0, _do, lambda f: f, fill)

                def _fl(f):
                    _flush(f)
                    return 0
                return lax.cond(fill > KB - 16, _fl, lambda f: f, fill)

            return lax.fori_loop(0, CH // 16, _grp, fill)

        fill = lax.fori_loop(0, NCH, _chunk, 0)

        def _fl2(f):
            _flush(f)
            return 0
        lax.cond(fill > 0, _fl2, lambda f: f, fill)

        plsc.subcore_barrier()

        obase = c * NP3 + plo + s * R2
        rem = R2 % KB
        for acc, out in ((acc0, s0_h), (acc1, s1_h), (acc2, s2_h)):
            for q in range(R2 // KB):
                pltpu.sync_copy(acc.at[pl.ds(s * R2 + q * KB, KB)], bufC)
                pltpu.sync_copy(bufC, out.at[pl.ds(obase + q * KB, KB)])
            if rem:
                pltpu.sync_copy(acc.at[pl.ds(s * R2 + (R2 // KB) * KB, rem)],
                                bufC.at[pl.ds(0, rem)])
                pltpu.sync_copy(bufC.at[pl.ds(0, rem)],
                                out.at[pl.ds(obase + (R2 // KB) * KB, rem)])
        pltpu.sync_copy(maxacc.at[pl.ds(0, R2)],
                        smax_h.at[pl.ds(obase, R2)])

        plsc.subcore_barrier()


_sc_out4 = tuple(jax.ShapeDtypeStruct((2 * NP3, HH), jnp.float32)
                 for _ in range(4))


@functools.cache
def _build_edge():
    mesh = plsc.VectorSubcoreMesh(core_axis_name="c", subcore_axis_name="s")

    @functools.partial(
        pl.kernel, mesh=mesh,
        compiler_params=pltpu.CompilerParams(needs_layout_passes=False, use_tc_tiling_on_sc=False),
        out_type=_sc_out4,
        scratch_types=[
        pltpu.VMEM((NP,), jnp.float32),
        pltpu.VMEM((NP,), jnp.float32),
        pltpu.VMEM((R2 + 1, HH), jnp.float32),
        pltpu.VMEM((CH,), jnp.int32),
        pltpu.VMEM((CH,), jnp.int32),
        pltpu.VMEM((KB,), jnp.int32),
        pltpu.VMEM((KB,), jnp.int32),
        pltpu.VMEM((KB,), jnp.int32),
        pltpu.VMEM((KB,), jnp.float32),
        pltpu.VMEM((KB,), jnp.float32),
        pltpu.VMEM((KB, HH), jnp.float32),
        pltpu.VMEM((KB, HH), jnp.float32),
        pltpu.VMEM((KB, HH), jnp.float32),
        pltpu.VMEM((KB, HH), jnp.float32),
            pltpu.VMEM_SHARED((ACC_ROWS, HH), jnp.float32),
            pltpu.VMEM_SHARED((ACC_ROWS, HH), jnp.float32),
            pltpu.VMEM_SHARED((ACC_ROWS, HH), jnp.float32),
            pltpu.SemaphoreType.DMA,
            pltpu.SemaphoreType.DMA,
            pltpu.SemaphoreType.DMA,
        ],
    )
    def k(A2_h, C2_h, src_h, dst_h, eig1_h, rabs_h, s0, s1, s2, sm, *rest):
        _sc_edge_body(A2_h, C2_h, src_h, dst_h, eig1_h, rabs_h, s0, s1, s2,
                      sm, *rest)

    return k


def _sc_edge(A2, C2, src, dst, eig1, rabs):
    return _build_edge()(A2, C2, src, dst, eig1, rabs)


BR = 1000


def _ln(x, g, b):
    m = jnp.mean(x, axis=-1, keepdims=True)
    v = jnp.mean((x - m) ** 2, axis=-1, keepdims=True)
    return (x - m) * lax.rsqrt(v + 1e-5) * g + b


def _mm(x, w):
    return lax.dot_general(x, w, (((1,), (1,)), ((), ())),
                           preferred_element_type=jnp.float32)


def _tk_stats_body(x_r, o_r):
    x = x_r[...]
    rows = [jnp.sum(x[a * 16:(a + 1) * 16], axis=0, keepdims=True)
            for a in range(3)]
    o_r[...] = jnp.concatenate(rows + [jnp.zeros((5, x.shape[1]),
                                                 jnp.float32)], axis=0)


def _tk_stats(x):
    return pl.pallas_call(
        _tk_stats_body,
        grid=(NP // 1024,),
        in_specs=[pl.BlockSpec((48, 1024), lambda i: (0, i))],
        out_specs=pl.BlockSpec((8, 1024), lambda i: (0, i)),
        out_shape=jax.ShapeDtypeStruct((8, NP), jnp.float32),
    )(x)


def _tk_in_body(x_r, w_r, b_r, g_r, bb_r, o_r):
    h = _mm(x_r[...], w_r[...]) + b_r[...]
    o_r[...] = jnp.maximum(_ln(h, g_r[...], bb_r[...]), 0.0)


def _tk_in(x, w, b, g, bb):
    return pl.pallas_call(
        _tk_in_body,
        grid=(N // BR,),
        in_specs=[
            pl.BlockSpec((BR, H), lambda i: (i, 0)),
            pl.BlockSpec((H, H), lambda i: (0, 0)),
            pl.BlockSpec((1, H), lambda i: (0, 0)),
            pl.BlockSpec((1, H), lambda i: (0, 0)),
            pl.BlockSpec((1, H), lambda i: (0, 0)),
        ],
        out_specs=pl.BlockSpec((BR, H), lambda i: (i, 0)),
        out_shape=jax.ShapeDtypeStruct((N, H), jnp.float32),
    )(x, w, b, g, bb)


def _tk_ab_body(h_r, ws_r, wd_r, a_r, b_r):
    a_r[...] = _mm(h_r[...], ws_r[...])
    b_r[...] = _mm(h_r[...], wd_r[...])


def _tk_ab(h, ws, wd):
    return pl.pallas_call(
        _tk_ab_body,
        grid=(N // BR,),
        in_specs=[
            pl.BlockSpec((BR, H), lambda i: (i, 0)),
            pl.BlockSpec((H, H), lambda i: (0, 0)),
            pl.BlockSpec((H, H), lambda i: (0, 0)),
        ],
        out_specs=[
            pl.BlockSpec((BR, H), lambda i: (i, 0)),
            pl.BlockSpec((BR, H), lambda i: (i, 0)),
        ],
        out_shape=[jax.ShapeDtypeStruct((N, H), jnp.float32),
                   jax.ShapeDtypeStruct((N, H), jnp.float32)],
    )(h, ws, wd)


ER = 2000


def _tk_c_body(ef_r, w0_r, b0_r, w1_r, b1_r, c0_r, c1_r):
    ef = ef_r[...]
    c0_r[...] = _mm(ef, w0_r[...]) + b0_r[...]
    c1_r[...] = _mm(ef, w1_r[...]) + b1_r[...]


def _tk_c(ef, w0, b0, w1, b1):
    return pl.pallas_call(
        _tk_c_body,
        grid=(E // ER,),
        in_specs=[
            pl.BlockSpec((ER, 16), lambda i: (i, 0)),
            pl.BlockSpec((H, 16), lambda i: (0, 0)),
            pl.BlockSpec((1, H), lambda i: (0, 0)),
            pl.BlockSpec((H, 16), lambda i: (0, 0)),
            pl.BlockSpec((1, H), lambda i: (0, 0)),
        ],
        out_specs=[
            pl.BlockSpec((ER, H), lambda i: (i, 0)),
            pl.BlockSpec((ER, H), lambda i: (i, 0)),
        ],
        out_shape=[jax.ShapeDtypeStruct((E, H), jnp.float32),
                   jax.ShapeDtypeStruct((E, H), jnp.float32)],
    )(ef, w0, b0, w1, b1)


def _tk_tower_body(h_r, b_r, s0_r, s1_r, s2_r, sm_r, st_r, wu_r, bu_r, o_r):
    h = h_r[...]
    B = b_r[...]
    s0 = jnp.concatenate([s0_r[0], s0_r[1]], axis=-1)
    s1 = jnp.concatenate([s1_r[0], s1_r[1]], axis=-1)
    s2 = jnp.concatenate([s2_r[0], s2_r[1]], axis=-1)
    smx = jnp.concatenate([sm_r[0], sm_r[1]], axis=-1)
    st = st_r[...]
    deg = st[:, 0:1]
    rabs = st[:, 1:2]
    rdw = st[:, 2:3]
    degs = jnp.maximum(deg, 1.0)
    absf = rabs + 1e-30
    sumav = rabs / absf
    sumdx = rdw / absf
    s_sum = s0 + deg * B
    s_mean = s_sum / degs
    s_max = jnp.where(deg > 0.0, smx + B, 0.0)
    s_av = s1 + sumav * B
    s_dx = jnp.abs(s2 + sumdx * B - h * sumdx)
    logd = jnp.log(degs + 1.0)
    amp = logd / DELTA
    att = DELTA / logd
    wu = wu_r[...]
    aggs = (s_mean, s_max, s_sum, s_av, s_dx)
    p0 = _mm(h, wu[:, 0:H])
    for a in range(5):
        p0 = p0 + _mm(aggs[a], wu[:, (1 + a) * H:(2 + a) * H])
    p1 = _mm(aggs[0], wu[:, 6 * H:7 * H])
    p2 = _mm(aggs[0], wu[:, 11 * H:12 * H])
    for a in range(1, 5):
        p1 = p1 + _mm(aggs[a], wu[:, (6 + a) * H:(7 + a) * H])
        p2 = p2 + _mm(aggs[a], wu[:, (11 + a) * H:(12 + a) * H])
    o_r[...] = p0 + amp * p1 + att * p2 + bu_r[...]


def _tk_tower(h, B, s0, s1, s2, sm, st, wu, bu):
    half = pl.BlockSpec((2, BR, HH), lambda i: (0, i, 0))
    return pl.pallas_call(
        _tk_tower_body,
        grid=(N // BR,),
        in_specs=[
            pl.BlockSpec((BR, H), lambda i: (i, 0)),
            pl.BlockSpec((BR, H), lambda i: (i, 0)),
            half, half, half, half,
            pl.BlockSpec((BR, 8), lambda i: (i, 0)),
            pl.BlockSpec((H, 16 * H), lambda i: (0, 0)),
            pl.BlockSpec((1, H), lambda i: (0, 0)),
        ],
        out_specs=pl.BlockSpec((BR, H), lambda i: (i, 0)),
        out_shape=jax.ShapeDtypeStruct((N, H), jnp.float32),
    )(h, B, s0, s1, s2, sm, st, wu, bu)


def _tk_post_body(tw_r, h_r, wm_r, bm_r, bg_r, bb_r, lg_r, lb_r, rs_r, o_r,
                  stats):
    i = pl.program_id(0)
    phase = i // (N // BR)

    @pl.when(phase == 0)
    def _p0():
        @pl.when(i == 0)
        def _z():
            stats[...] = jnp.zeros((8, H), jnp.float32)
        tw = tw_r[...]
        stats[0:1, :] = stats[0:1, :] + jnp.sum(tw, axis=0, keepdims=True)
        stats[1:2, :] = stats[1:2, :] + jnp.sum(tw * tw, axis=0, keepdims=True)

    @pl.when(phase == 1)
    def _p1():
        tw = tw_r[...]
        h = h_r[...]
        mu = stats[0:1, :] / N
        var = stats[1:2, :] / N - mu * mu
        t = (tw - mu) * lax.rsqrt(var + 1e-5) * bg_r[...] + bb_r[...]
        mixed = _mm(t, wm_r[...]) + bm_r[...]
        mixed = jnp.where(mixed > 0.0, mixed, 0.01 * mixed)
        conv = mixed + h
        hn = jnp.maximum(_ln(conv, lg_r[...], lb_r[...]), 0.0)
        gate = 1.0 / (1.0 + jnp.exp(-rs_r[...]))
        o_r[...] = hn + gate * h


def _tk_post(tw, h, wm, bm, bg, bb, lg, lb, rs):
    nb = N // BR
    return pl.pallas_call(
        _tk_post_body,
        grid=(2 * nb,),
        in_specs=[
            pl.BlockSpec((BR, H), lambda i: (i % nb, 0)),
            pl.BlockSpec((BR, H), lambda i: (i % nb, 0)),
            pl.BlockSpec((H, H), lambda i: (0, 0)),
            pl.BlockSpec((1, H), lambda i: (0, 0)),
            pl.BlockSpec((1, H), lambda i: (0, 0)),
            pl.BlockSpec((1, H), lambda i: (0, 0)),
            pl.BlockSpec((1, H), lambda i: (0, 0)),
            pl.BlockSpec((1, H), lambda i: (0, 0)),
            pl.BlockSpec((1, H), lambda i: (0, 0)),
        ],
        out_specs=pl.BlockSpec((BR, H), lambda i: (i % nb, 0)),
        out_shape=jax.ShapeDtypeStruct((N, H), jnp.float32),
        scratch_shapes=[pltpu.VMEM((8, H), jnp.float32)],
    )(tw, h, wm, bm, bg, bb, lg, lb, rs)


def _tk_out_body(h_r, w1_r, b1_r, g_r, bb_r, w2_r, b2_r, o_r):
    t = _mm(h_r[...], w1_r[...]) + b1_r[...]
    t = jnp.maximum(_ln(t, g_r[...], bb_r[...]), 0.0)
    o_r[...] = _mm(t, w2_r[...]) + b2_r[...]


def _tk_out(h, w1, b1, g, bb, w2, b2):
    return pl.pallas_call(
        _tk_out_body,
        grid=(N // BR,),
        in_specs=[
            pl.BlockSpec((BR, H), lambda i: (i, 0)),
            pl.BlockSpec((H, H), lambda i: (0, 0)),
            pl.BlockSpec((1, H), lambda i: (0, 0)),
            pl.BlockSpec((1, H), lambda i: (0, 0)),
            pl.BlockSpec((1, H), lambda i: (0, 0)),
            pl.BlockSpec((H, H), lambda i: (0, 0)),
            pl.BlockSpec((1, H), lambda i: (0, 0)),
        ],
        out_specs=pl.BlockSpec((BR, H), lambda i: (i, 0)),
        out_shape=jax.ShapeDtypeStruct((N, H), jnp.float32),
    )(h, w1, b1, g, bb, w2, b2)


def kernel(node_features, edge_index, edge_feat, eig_vec, W_in, b_in, ln_in_g,
           ln_in_b, W_M, b_M, W_U, b_U, bn_g, bn_b, W_mix, b_mix, ln_int_g,
           ln_int_b, res_scale, W_o1, b_o1, ln_o_g, ln_o_b, W_o2, b_o2):
    src = edge_index[0]
    dst = edge_index[1]
    eig1 = jnp.zeros((NP,), jnp.float32).at[:N].set(eig_vec[:, 1])

    r2 = lambda v: v.reshape(1, H)
    h = _tk_in(node_features, W_in, r2(b_in), r2(ln_in_g), r2(ln_in_b))

    parts = _sc_pass0(dst, src, eig1).reshape(48, NP)
    stats3 = _tk_stats(parts)
    st = jnp.zeros((N, 8), jnp.float32).at[:, 0:3].set(stats3[:3, :N].T)

    C0, C1 = _tk_c(edge_feat, W_M[0][:, 2 * H:], r2(b_M[0]),
                   W_M[1][:, 2 * H:], r2(b_M[1]))
    Cs = (C0, C1)

    for l in range(2):
        A, B = _tk_ab(h, W_M[l][:, :H], W_M[l][:, H:2 * H])
        A2 = A.reshape(2 * N, HH)
        C2 = Cs[l].reshape(2 * E, HH)
        s0, s1, s2, sm = _sc_edge(A2, C2, src, dst, eig1, stats3[1])
        r3 = lambda x: x.reshape(2, NP3, HH)[:, :N]
        tower = _tk_tower(h, B, r3(s0), r3(s1), r3(s2), r3(sm),
                          st, W_U[l], r2(b_U[l]))
        h = _tk_post(tower, h, W_mix[l], r2(b_mix[l]), r2(bn_g[l]),
                     r2(bn_b[l]), r2(ln_int_g[l]), r2(ln_int_b[l]),
                     jnp.full((1, H), res_scale[l], jnp.float32))

    return _tk_out(h, W_o1, r2(b_o1), r2(ln_o_g), r2(ln_o_b), W_o2, r2(b_o2))
